# baseline (speedup 1.0000x reference)
import functools

import jax
import jax.numpy as jnp
from jax import lax
from jax.experimental import pallas as pl
from jax.experimental.pallas import tpu as pltpu
from jax.experimental.pallas import tpu_sc as plsc

NUM_CLASSES = 100000
FEAT_DIM = 512
BATCH = 16384

NC = 2
NS = 16
LANES = 16
NW = NC * NS
BPW = BATCH // NW
ROW_CHUNK = 64
NCHUNK = BPW // ROW_CHUNK
NVEC = FEAT_DIM // LANES


def _sc_partials(x, labels_i32, centers):
  mesh = plsc.VectorSubcoreMesh(
      core_axis_name="c", subcore_axis_name="s", num_cores=NC,
      num_subcores=NS)

  out_t = jax.ShapeDtypeStruct((BATCH, LANES), jnp.float32)

  @functools.partial(
      pl.kernel,
      out_type=[out_t, out_t, out_t],
      mesh=mesh,
      compiler_params=pltpu.CompilerParams(use_tc_tiling_on_sc=False),
      scratch_types=[
          pltpu.VMEM((ROW_CHUNK,), jnp.int32),
          pltpu.VMEM((ROW_CHUNK, FEAT_DIM), jnp.float32),
          pltpu.VMEM((ROW_CHUNK, FEAT_DIM), jnp.float32),
          pltpu.VMEM((BPW, LANES), jnp.float32),
          pltpu.VMEM((BPW, LANES), jnp.float32),
          pltpu.VMEM((BPW, LANES), jnp.float32),
          pltpu.SemaphoreType.DMA,
      ],
  )
  def sc_kernel(x_hbm, lab_hbm, cen_hbm, outd, outc, outx,
                idx_v, x_v, c_v, ad_v, ac_v, ax_v, sem):
    wid = lax.axis_index("s") * NC + lax.axis_index("c")
    base = wid * BPW
    for k in range(NCHUNK):
      off = base + k * ROW_CHUNK
      pltpu.sync_copy(lab_hbm.at[pl.ds(off, ROW_CHUNK)], idx_v)
      pltpu.sync_copy(x_hbm.at[pl.ds(off, ROW_CHUNK)], x_v)
      pltpu.async_copy(cen_hbm.at[idx_v], c_v, sem).wait()

      @plsc.parallel_loop(0, ROW_CHUNK, unroll=2)
      def row_body(r):
        zero = jnp.zeros((LANES,), jnp.float32)
        pd = [zero, zero, zero, zero]
        pc = [zero, zero, zero, zero]
        px = [zero, zero, zero, zero]
        for v in range(NVEC):
          xv = x_v[r, pl.ds(v * LANES, LANES)]
          cv = c_v[r, pl.ds(v * LANES, LANES)]
          j = v % 4
          pd[j] = pd[j] + xv * cv
          pc[j] = pc[j] + cv * cv
          px[j] = px[j] + xv * xv
        row = k * ROW_CHUNK + r
        ad_v[row, :] = (pd[0] + pd[1]) + (pd[2] + pd[3])
        ac_v[row, :] = (pc[0] + pc[1]) + (pc[2] + pc[3])
        ax_v[row, :] = (px[0] + px[1]) + (px[2] + px[3])

    pltpu.sync_copy(ad_v, outd.at[pl.ds(base, BPW)])
    pltpu.sync_copy(ac_v, outc.at[pl.ds(base, BPW)])
    pltpu.sync_copy(ax_v, outx.at[pl.ds(base, BPW)])

  return sc_kernel(x, labels_i32, centers)


def _tc_finish(dp, cp, xp):
  block = 2048
  grid = BATCH // block

  def body(d_ref, c_ref, x_ref, o_ref):
    i = pl.program_id(0)
    dot = jnp.sum(d_ref[...], axis=1)
    c2 = jnp.sum(c_ref[...], axis=1)
    x2 = jnp.sum(x_ref[...], axis=1)
    cos = dot / jnp.maximum(jnp.sqrt(x2) * jnp.sqrt(c2), 1e-8)
    dist = jnp.exp(-3.5 * (cos - 1.0))
    s = jnp.sum(dist) * (1.0 / BATCH)

    @pl.when(i == 0)
    def _():
      o_ref[0, 0] = 0.0

    o_ref[0, 0] += s

  out = pl.pallas_call(
      body,
      grid=(grid,),
      in_specs=[pl.BlockSpec((block, LANES), lambda i: (i, 0))] * 3,
      out_specs=pl.BlockSpec(memory_space=pltpu.SMEM),
      out_shape=jax.ShapeDtypeStruct((1, 1), jnp.float32),
  )(dp, cp, xp)
  return out[0, 0]


@jax.jit
def kernel(x, labels, centers):
  labels_i32 = labels.astype(jnp.int32)
  dp, cp, xp = _sc_partials(x, labels_i32, centers)
  return _tc_finish(dp, cp, xp)


# speedup vs baseline: 3.1217x; 3.1217x over previous
import functools

import jax
import jax.numpy as jnp
from jax import lax
from jax.experimental import pallas as pl
from jax.experimental.pallas import tpu as pltpu
from jax.experimental.pallas import tpu_sc as plsc

NUM_CLASSES = 100000
FEAT_DIM = 512
BATCH = 16384

NC = 2
NS = 16
LANES = 16
NW = NC * NS
BPW = BATCH // NW
ROW_CHUNK = 64
NCHUNK = BPW // ROW_CHUNK
NVEC = FEAT_DIM // LANES
OUT_ROWS = BATCH // 8


def _sc_partials(x, labels_i32, centers):
  mesh = plsc.VectorSubcoreMesh(
      core_axis_name="c", subcore_axis_name="s", num_cores=NC,
      num_subcores=NS)

  out_t = jax.ShapeDtypeStruct((OUT_ROWS, 128), jnp.float32)

  @functools.partial(
      pl.kernel,
      out_type=[out_t, out_t, out_t],
      mesh=mesh,
      compiler_params=pltpu.CompilerParams(use_tc_tiling_on_sc=True),
      scratch_types=[
          pltpu.VMEM((ROW_CHUNK,), jnp.int32),
          pltpu.VMEM((ROW_CHUNK, FEAT_DIM), jnp.float32),
          pltpu.VMEM((ROW_CHUNK, FEAT_DIM), jnp.float32),
          pltpu.VMEM((ROW_CHUNK // 8, 128), jnp.float32),
          pltpu.VMEM((ROW_CHUNK // 8, 128), jnp.float32),
          pltpu.VMEM((ROW_CHUNK // 8, 128), jnp.float32),
          pltpu.SemaphoreType.DMA,
      ],
  )
  def sc_kernel(x_hbm, lab_hbm, cen_hbm, outd, outc, outx,
                idx_v, x_v, c_v, ad_v, ac_v, ax_v, sem):
    wid = lax.axis_index("s") * NC + lax.axis_index("c")
    base = wid * BPW
    for k in range(NCHUNK):
      off = pl.multiple_of(base + k * ROW_CHUNK, ROW_CHUNK)
      pltpu.sync_copy(lab_hbm.at[pl.ds(off, ROW_CHUNK)], idx_v)
      pltpu.sync_copy(x_hbm.at[pl.ds(off, ROW_CHUNK)], x_v)
      pltpu.async_copy(cen_hbm.at[idx_v], c_v, sem).wait()

      @plsc.parallel_loop(0, ROW_CHUNK, unroll=2)
      def row_body(r):
        zero = jnp.zeros((LANES,), jnp.float32)
        pd = [zero, zero, zero, zero]
        pc = [zero, zero, zero, zero]
        px = [zero, zero, zero, zero]
        for v in range(NVEC):
          xv = x_v[r, pl.ds(v * LANES, LANES)]
          cv = c_v[r, pl.ds(v * LANES, LANES)]
          j = v % 4
          pd[j] = pd[j] + xv * cv
          pc[j] = pc[j] + cv * cv
          px[j] = px[j] + xv * xv
        q = r // 8
        s = (r % 8) * LANES
        ad_v[q, pl.ds(s, LANES)] = (pd[0] + pd[1]) + (pd[2] + pd[3])
        ac_v[q, pl.ds(s, LANES)] = (pc[0] + pc[1]) + (pc[2] + pc[3])
        ax_v[q, pl.ds(s, LANES)] = (px[0] + px[1]) + (px[2] + px[3])

      orow = pl.multiple_of(off // 8, 8)
      pltpu.sync_copy(ad_v, outd.at[pl.ds(orow, ROW_CHUNK // 8)])
      pltpu.sync_copy(ac_v, outc.at[pl.ds(orow, ROW_CHUNK // 8)])
      pltpu.sync_copy(ax_v, outx.at[pl.ds(orow, ROW_CHUNK // 8)])

  return sc_kernel(x, labels_i32, centers)


def _tc_finish(dp, cp, xp):
  block = 256
  grid = OUT_ROWS // block

  def body(d_ref, c_ref, x_ref, o_ref):
    i = pl.program_id(0)
    sel = (lax.broadcasted_iota(jnp.int32, (128, 8), 0) // LANES
           == lax.broadcasted_iota(jnp.int32, (128, 8), 1)
           ).astype(jnp.float32)
    dot = lax.dot_general(d_ref[...], sel, (((1,), (0,)), ((), ())),
                          preferred_element_type=jnp.float32)
    c2 = lax.dot_general(c_ref[...], sel, (((1,), (0,)), ((), ())),
                         preferred_element_type=jnp.float32)
    x2 = lax.dot_general(x_ref[...], sel, (((1,), (0,)), ((), ())),
                         preferred_element_type=jnp.float32)
    cos = dot / jnp.maximum(jnp.sqrt(x2) * jnp.sqrt(c2), 1e-8)
    dist = jnp.exp(-3.5 * (cos - 1.0))
    s = jnp.sum(dist) * (1.0 / BATCH)

    @pl.when(i == 0)
    def _():
      o_ref[0, 0] = 0.0

    o_ref[0, 0] += s

  out = pl.pallas_call(
      body,
      grid=(grid,),
      in_specs=[pl.BlockSpec((block, 128), lambda i: (i, 0))] * 3,
      out_specs=pl.BlockSpec(memory_space=pltpu.SMEM),
      out_shape=jax.ShapeDtypeStruct((1, 1), jnp.float32),
  )(dp, cp, xp)
  return out[0, 0]


@jax.jit
def kernel(x, labels, centers):
  labels_i32 = labels.astype(jnp.int32)
  dp, cp, xp = _sc_partials(x, labels_i32, centers)
  return _tc_finish(dp, cp, xp)


# speedup vs baseline: 4.1925x; 1.3430x over previous
import functools

import jax
import jax.numpy as jnp
from jax import lax
from jax.experimental import pallas as pl
from jax.experimental.pallas import tpu as pltpu
from jax.experimental.pallas import tpu_sc as plsc

NUM_CLASSES = 100000
FEAT_DIM = 512
BATCH = 16384

NC = 2
NS = 16
LANES = 16
NW = NC * NS
BPW = BATCH // NW
ROW_CHUNK = 32
NCHUNK = BPW // ROW_CHUNK
NBUF = 2
NVEC = FEAT_DIM // LANES
OUT_ROWS = BATCH // 8


def _sc_partials(x, labels_i32, centers):
  mesh = plsc.VectorSubcoreMesh(
      core_axis_name="c", subcore_axis_name="s", num_cores=NC,
      num_subcores=NS)

  out_t = jax.ShapeDtypeStruct((OUT_ROWS, 128), jnp.float32)

  @functools.partial(
      pl.kernel,
      out_type=[out_t, out_t, out_t],
      mesh=mesh,
      compiler_params=pltpu.CompilerParams(use_tc_tiling_on_sc=True),
      scratch_types=[
          pltpu.VMEM((BPW,), jnp.int32),
          pltpu.VMEM((NBUF, ROW_CHUNK, FEAT_DIM), jnp.float32),
          pltpu.VMEM((NBUF, ROW_CHUNK, FEAT_DIM), jnp.float32),
          pltpu.VMEM((BPW // 8, 128), jnp.float32),
          pltpu.VMEM((BPW // 8, 128), jnp.float32),
          pltpu.VMEM((BPW // 8, 128), jnp.float32),
          pltpu.SemaphoreType.DMA,
          pltpu.SemaphoreType.DMA,
          pltpu.SemaphoreType.DMA,
          pltpu.SemaphoreType.DMA,
          pltpu.SemaphoreType.DMA,
          pltpu.SemaphoreType.DMA,
      ],
  )
  def sc_kernel(x_hbm, lab_hbm, cen_hbm, outd, outc, outx,
                idx_v, x_bufs, c_bufs, ad_v, ac_v, ax_v, *sems):
    semx = sems[:NBUF]
    semc = sems[NBUF:]
    wid = lax.axis_index("s") * NC + lax.axis_index("c")
    base = pl.multiple_of(wid * BPW, BPW)
    pltpu.sync_copy(lab_hbm.at[pl.ds(base, BPW)], idx_v)

    def start(k):
      b = k % NBUF
      off = pl.multiple_of(base + k * ROW_CHUNK, ROW_CHUNK)
      dx = pltpu.async_copy(
          x_hbm.at[pl.ds(off, ROW_CHUNK)], x_bufs.at[b], semx[b])
      dc = pltpu.async_copy(
          cen_hbm.at[idx_v.at[pl.ds(k * ROW_CHUNK, ROW_CHUNK)]],
          c_bufs.at[b], semc[b])
      return dx, dc

    descs = {}
    for k in range(min(NBUF, NCHUNK)):
      descs[k] = start(k)

    for k in range(NCHUNK):
      b = k % NBUF
      dx, dc = descs.pop(k)
      dx.wait()
      dc.wait()
      x_v = x_bufs.at[b]
      c_v = c_bufs.at[b]

      @plsc.parallel_loop(0, ROW_CHUNK, unroll=2)
      def row_body(r):
        zero = jnp.zeros((LANES,), jnp.float32)
        pd = [zero, zero, zero, zero]
        pc = [zero, zero, zero, zero]
        px = [zero, zero, zero, zero]
        for v in range(NVEC):
          xv = x_v[r, pl.ds(v * LANES, LANES)]
          cv = c_v[r, pl.ds(v * LANES, LANES)]
          j = v % 4
          pd[j] = pd[j] + xv * cv
          pc[j] = pc[j] + cv * cv
          px[j] = px[j] + xv * xv
        q = k * (ROW_CHUNK // 8) + r // 8
        s = (r % 8) * LANES
        ad_v[q, pl.ds(s, LANES)] = (pd[0] + pd[1]) + (pd[2] + pd[3])
        ac_v[q, pl.ds(s, LANES)] = (pc[0] + pc[1]) + (pc[2] + pc[3])
        ax_v[q, pl.ds(s, LANES)] = (px[0] + px[1]) + (px[2] + px[3])

      if k + NBUF < NCHUNK:
        descs[k + NBUF] = start(k + NBUF)

    orow = pl.multiple_of(base // 8, 8)
    pltpu.sync_copy(ad_v, outd.at[pl.ds(orow, BPW // 8)])
    pltpu.sync_copy(ac_v, outc.at[pl.ds(orow, BPW // 8)])
    pltpu.sync_copy(ax_v, outx.at[pl.ds(orow, BPW // 8)])

  return sc_kernel(x, labels_i32, centers)


def _tc_finish(dp, cp, xp):
  block = 256
  grid = OUT_ROWS // block

  def body(d_ref, c_ref, x_ref, o_ref):
    i = pl.program_id(0)
    sel = (lax.broadcasted_iota(jnp.int32, (128, 8), 0) // LANES
           == lax.broadcasted_iota(jnp.int32, (128, 8), 1)
           ).astype(jnp.float32)
    dot = lax.dot_general(d_ref[...], sel, (((1,), (0,)), ((), ())),
                          preferred_element_type=jnp.float32)
    c2 = lax.dot_general(c_ref[...], sel, (((1,), (0,)), ((), ())),
                         preferred_element_type=jnp.float32)
    x2 = lax.dot_general(x_ref[...], sel, (((1,), (0,)), ((), ())),
                         preferred_element_type=jnp.float32)
    cos = dot / jnp.maximum(jnp.sqrt(x2) * jnp.sqrt(c2), 1e-8)
    dist = jnp.exp(-3.5 * (cos - 1.0))
    s = jnp.sum(dist) * (1.0 / BATCH)

    @pl.when(i == 0)
    def _():
      o_ref[0, 0] = 0.0

    o_ref[0, 0] += s

  out = pl.pallas_call(
      body,
      grid=(grid,),
      in_specs=[pl.BlockSpec((block, 128), lambda i: (i, 0))] * 3,
      out_specs=pl.BlockSpec(memory_space=pltpu.SMEM),
      out_shape=jax.ShapeDtypeStruct((1, 1), jnp.float32),
  )(dp, cp, xp)
  return out[0, 0]


@jax.jit
def kernel(x, labels, centers):
  labels_i32 = labels.astype(jnp.int32)
  dp, cp, xp = _sc_partials(x, labels_i32, centers)
  return _tc_finish(dp, cp, xp)


# speedup vs baseline: 4.3882x; 1.0467x over previous
import functools

import jax
import jax.numpy as jnp
from jax import lax
from jax.experimental import pallas as pl
from jax.experimental.pallas import tpu as pltpu
from jax.experimental.pallas import tpu_sc as plsc

NUM_CLASSES = 100000
FEAT_DIM = 512
BATCH = 16384

NC = 2
NS = 16
LANES = 16
NW = NC * NS
BPW = BATCH // NW
ROW_CHUNK = 32
NCHUNK = BPW // ROW_CHUNK
NBUF = 2
NVEC = FEAT_DIM // LANES
OUT_ROWS = BATCH // 8


def _sc_partials(x, labels_i32, centers):
  mesh = plsc.VectorSubcoreMesh(
      core_axis_name="c", subcore_axis_name="s", num_cores=NC,
      num_subcores=NS)

  out_t = jax.ShapeDtypeStruct((OUT_ROWS, 128), jnp.float32)

  @functools.partial(
      pl.kernel,
      out_type=[out_t, out_t, out_t],
      mesh=mesh,
      compiler_params=pltpu.CompilerParams(use_tc_tiling_on_sc=True),
      scratch_types=[
          pltpu.VMEM((BPW,), jnp.int32),
          pltpu.VMEM((NBUF, ROW_CHUNK, FEAT_DIM), jnp.float32),
          pltpu.VMEM((NBUF, ROW_CHUNK, FEAT_DIM), jnp.float32),
          pltpu.VMEM((BPW // 8, 128), jnp.float32),
          pltpu.VMEM((BPW // 8, 128), jnp.float32),
          pltpu.VMEM((BPW // 8, 128), jnp.float32),
          pltpu.SemaphoreType.DMA,
          pltpu.SemaphoreType.DMA,
          pltpu.SemaphoreType.DMA,
          pltpu.SemaphoreType.DMA,
          pltpu.SemaphoreType.DMA,
          pltpu.SemaphoreType.DMA,
      ],
  )
  def sc_kernel(x_hbm, lab_hbm, cen_hbm, outd, outc, outx,
                idx_v, x_bufs, c_bufs, ad_v, ac_v, ax_v, *sems):
    semx = sems[:NBUF]
    semc = sems[NBUF:]
    wid = lax.axis_index("s") * NC + lax.axis_index("c")
    base = pl.multiple_of(wid * BPW, BPW)
    pltpu.sync_copy(lab_hbm.at[pl.ds(base, BPW)], idx_v)

    def start(k):
      b = k % NBUF
      off = pl.multiple_of(base + k * ROW_CHUNK, ROW_CHUNK)
      dx = pltpu.async_copy(
          x_hbm.at[pl.ds(off, ROW_CHUNK)], x_bufs.at[b], semx[b])
      dc = pltpu.async_copy(
          cen_hbm.at[idx_v.at[pl.ds(k * ROW_CHUNK, ROW_CHUNK)]],
          c_bufs.at[b], semc[b])
      return dx, dc

    descs = {}
    for k in range(min(NBUF, NCHUNK)):
      descs[k] = start(k)

    for k in range(NCHUNK):
      b = k % NBUF
      dx, dc = descs.pop(k)
      dx.wait()
      dc.wait()
      x_v = x_bufs.at[b]
      c_v = c_bufs.at[b]

      @plsc.parallel_loop(0, ROW_CHUNK, unroll=2)
      def row_body(r):
        zero = jnp.zeros((LANES,), jnp.float32)
        pd = [zero, zero, zero, zero]
        pc = [zero, zero, zero, zero]
        px = [zero, zero, zero, zero]
        for v in range(NVEC):
          xv = x_v[r, pl.ds(v * LANES, LANES)]
          cv = c_v[r, pl.ds(v * LANES, LANES)]
          j = v % 4
          pd[j] = pd[j] + xv * cv
          pc[j] = pc[j] + cv * cv
          px[j] = px[j] + xv * xv
        q = k * (ROW_CHUNK // 8) + r // 8
        s = (r % 8) * LANES
        ad_v[q, pl.ds(s, LANES)] = (pd[0] + pd[1]) + (pd[2] + pd[3])
        ac_v[q, pl.ds(s, LANES)] = (pc[0] + pc[1]) + (pc[2] + pc[3])
        ax_v[q, pl.ds(s, LANES)] = (px[0] + px[1]) + (px[2] + px[3])

      if k + NBUF < NCHUNK:
        descs[k + NBUF] = start(k + NBUF)

    orow = pl.multiple_of(base // 8, 8)
    pltpu.sync_copy(ad_v, outd.at[pl.ds(orow, BPW // 8)])
    pltpu.sync_copy(ac_v, outc.at[pl.ds(orow, BPW // 8)])
    pltpu.sync_copy(ax_v, outx.at[pl.ds(orow, BPW // 8)])

  return sc_kernel(x, labels_i32, centers)


def _tc_finish(dp, cp, xp, sel):
  def body(d_ref, c_ref, x_ref, sel_ref, o_ref):
    sel_m = sel_ref[...]
    dot = lax.dot_general(d_ref[...], sel_m, (((1,), (0,)), ((), ())),
                          preferred_element_type=jnp.float32)
    c2 = lax.dot_general(c_ref[...], sel_m, (((1,), (0,)), ((), ())),
                         preferred_element_type=jnp.float32)
    x2 = lax.dot_general(x_ref[...], sel_m, (((1,), (0,)), ((), ())),
                         preferred_element_type=jnp.float32)
    cos = dot / jnp.maximum(jnp.sqrt(x2) * jnp.sqrt(c2), 1e-8)
    dist = jnp.exp(-3.5 * (cos - 1.0))
    o_ref[0, 0] = jnp.sum(dist) * (1.0 / BATCH)

  out = pl.pallas_call(
      body,
      in_specs=[pl.BlockSpec((OUT_ROWS, 128), lambda: (0, 0))] * 3
      + [pl.BlockSpec((128, 8), lambda: (0, 0))],
      out_specs=pl.BlockSpec(memory_space=pltpu.SMEM),
      out_shape=jax.ShapeDtypeStruct((1, 1), jnp.float32),
  )(dp, cp, xp, sel)
  return out[0, 0]


@jax.jit
def kernel(x, labels, centers):
  labels_i32 = labels.astype(jnp.int32)
  dp, cp, xp = _sc_partials(x, labels_i32, centers)
  sel = (jnp.arange(128, dtype=jnp.int32)[:, None] // LANES
         == jnp.arange(8, dtype=jnp.int32)[None, :]).astype(jnp.float32)
  return _tc_finish(dp, cp, xp, sel)


# speedup vs baseline: 4.7007x; 1.0712x over previous
import functools

import jax
import jax.numpy as jnp
from jax import lax
from jax.experimental import pallas as pl
from jax.experimental.pallas import tpu as pltpu
from jax.experimental.pallas import tpu_sc as plsc

NUM_CLASSES = 100000
FEAT_DIM = 512
BATCH = 16384

NC = 2
NS = 16
LANES = 16
NW = NC * NS
BPW = BATCH // NW
ROW_CHUNK = 32
NCHUNK = BPW // ROW_CHUNK
NBUF = 2
NVEC = FEAT_DIM // LANES
OUT_ROWS = BATCH // 8


def _sc_partials(x, labels_i32, centers):
  mesh = plsc.VectorSubcoreMesh(
      core_axis_name="c", subcore_axis_name="s", num_cores=NC,
      num_subcores=NS)

  out_t = jax.ShapeDtypeStruct((OUT_ROWS, 128), jnp.float32)

  @functools.partial(
      pl.kernel,
      out_type=[out_t, out_t, out_t],
      mesh=mesh,
      compiler_params=pltpu.CompilerParams(use_tc_tiling_on_sc=True),
      scratch_types=[
          pltpu.VMEM((BPW,), jnp.int32),
          pltpu.VMEM((NBUF, ROW_CHUNK, FEAT_DIM), jnp.float32),
          pltpu.VMEM((NBUF, ROW_CHUNK, FEAT_DIM), jnp.float32),
          pltpu.VMEM((BPW // 8, 128), jnp.float32),
          pltpu.VMEM((BPW // 8, 128), jnp.float32),
          pltpu.VMEM((BPW // 8, 128), jnp.float32),
          pltpu.SemaphoreType.DMA,
          pltpu.SemaphoreType.DMA,
          pltpu.SemaphoreType.DMA,
          pltpu.SemaphoreType.DMA,
          pltpu.SemaphoreType.DMA,
          pltpu.SemaphoreType.DMA,
      ],
  )
  def sc_kernel(x_hbm, lab_hbm, cen_hbm, outd, outc, outx,
                idx_v, x_bufs, c_bufs, ad_v, ac_v, ax_v, *sems):
    semx = sems[:NBUF]
    semc = sems[NBUF:]
    wid = lax.axis_index("s") * NC + lax.axis_index("c")
    base = pl.multiple_of(wid * BPW, BPW)
    pltpu.sync_copy(lab_hbm.at[pl.ds(base, BPW)], idx_v)

    def start(k):
      b = k % NBUF
      off = pl.multiple_of(base + k * ROW_CHUNK, ROW_CHUNK)
      dx = pltpu.async_copy(
          x_hbm.at[pl.ds(off, ROW_CHUNK)], x_bufs.at[b], semx[b])
      dc = pltpu.async_copy(
          cen_hbm.at[idx_v.at[pl.ds(k * ROW_CHUNK, ROW_CHUNK)]],
          c_bufs.at[b], semc[b])
      return dx, dc

    descs = {}
    for k in range(min(NBUF, NCHUNK)):
      descs[k] = start(k)

    for k in range(NCHUNK):
      b = k % NBUF
      dx, dc = descs.pop(k)
      dx.wait()
      dc.wait()
      x_v = x_bufs.at[b]
      c_v = c_bufs.at[b]

      @plsc.parallel_loop(0, ROW_CHUNK, unroll=4)
      def row_body(r):
        zero = jnp.zeros((LANES,), jnp.float32)
        pd = [zero, zero]
        pc = [zero, zero]
        px = [zero, zero]
        for v in range(NVEC):
          xv = x_v[r, pl.ds(v * LANES, LANES)]
          cv = c_v[r, pl.ds(v * LANES, LANES)]
          j = v % 2
          pd[j] = pd[j] + xv * cv
          pc[j] = pc[j] + cv * cv
          px[j] = px[j] + xv * xv
        q = k * (ROW_CHUNK // 8) + r // 8
        s = (r % 8) * LANES
        ad_v[q, pl.ds(s, LANES)] = pd[0] + pd[1]
        ac_v[q, pl.ds(s, LANES)] = pc[0] + pc[1]
        ax_v[q, pl.ds(s, LANES)] = px[0] + px[1]

      if k + NBUF < NCHUNK:
        descs[k + NBUF] = start(k + NBUF)

    orow = pl.multiple_of(base // 8, 8)
    pltpu.sync_copy(ad_v, outd.at[pl.ds(orow, BPW // 8)])
    pltpu.sync_copy(ac_v, outc.at[pl.ds(orow, BPW // 8)])
    pltpu.sync_copy(ax_v, outx.at[pl.ds(orow, BPW // 8)])

  return sc_kernel(x, labels_i32, centers)


def _tc_finish(dp, cp, xp, sel):
  def body(d_ref, c_ref, x_ref, sel_ref, o_ref):
    sel_m = sel_ref[...]
    dot = lax.dot_general(d_ref[...], sel_m, (((1,), (0,)), ((), ())),
                          preferred_element_type=jnp.float32)
    c2 = lax.dot_general(c_ref[...], sel_m, (((1,), (0,)), ((), ())),
                         preferred_element_type=jnp.float32)
    x2 = lax.dot_general(x_ref[...], sel_m, (((1,), (0,)), ((), ())),
                         preferred_element_type=jnp.float32)
    cos = dot / jnp.maximum(jnp.sqrt(x2) * jnp.sqrt(c2), 1e-8)
    dist = jnp.exp(-3.5 * (cos - 1.0))
    o_ref[0, 0] = jnp.sum(dist) * (1.0 / BATCH)

  out = pl.pallas_call(
      body,
      in_specs=[pl.BlockSpec((OUT_ROWS, 128), lambda: (0, 0))] * 3
      + [pl.BlockSpec((128, 8), lambda: (0, 0))],
      out_specs=pl.BlockSpec(memory_space=pltpu.SMEM),
      out_shape=jax.ShapeDtypeStruct((1, 1), jnp.float32),
  )(dp, cp, xp, sel)
  return out[0, 0]


@jax.jit
def kernel(x, labels, centers):
  labels_i32 = labels.astype(jnp.int32)
  dp, cp, xp = _sc_partials(x, labels_i32, centers)
  sel = (jnp.arange(128, dtype=jnp.int32)[:, None] // LANES
         == jnp.arange(8, dtype=jnp.int32)[None, :]).astype(jnp.float32)
  return _tc_finish(dp, cp, xp, sel)


# speedup vs baseline: 5.2231x; 1.1111x over previous
import functools

import jax
import jax.numpy as jnp
from jax import lax
from jax.experimental import pallas as pl
from jax.experimental.pallas import tpu as pltpu
from jax.experimental.pallas import tpu_sc as plsc

NUM_CLASSES = 100000
FEAT_DIM = 512
BATCH = 16384

NC = 2
NS = 16
LANES = 16
NW = NC * NS
BPW = BATCH // NW
ROW_CHUNK = 32
NCHUNK = BPW // ROW_CHUNK
NBUF = 2
NVEC = FEAT_DIM // LANES
OUT_ROWS = BATCH // 8


def _sc_partials(x, labels_i32, centers):
  mesh = plsc.VectorSubcoreMesh(
      core_axis_name="c", subcore_axis_name="s", num_cores=NC,
      num_subcores=NS)

  out_t = jax.ShapeDtypeStruct((OUT_ROWS, 128), jnp.float32)

  @functools.partial(
      pl.kernel,
      out_type=[out_t, out_t, out_t],
      mesh=mesh,
      compiler_params=pltpu.CompilerParams(use_tc_tiling_on_sc=True),
      scratch_types=[
          pltpu.VMEM((BPW,), jnp.int32),
          pltpu.VMEM((NBUF, ROW_CHUNK, FEAT_DIM), jnp.float32),
          pltpu.VMEM((NBUF, ROW_CHUNK, FEAT_DIM), jnp.float32),
          pltpu.VMEM((BPW // 8, 128), jnp.float32),
          pltpu.VMEM((BPW // 8, 128), jnp.float32),
          pltpu.VMEM((BPW // 8, 128), jnp.float32),
          pltpu.SemaphoreType.DMA,
          pltpu.SemaphoreType.DMA,
          pltpu.SemaphoreType.DMA,
          pltpu.SemaphoreType.DMA,
          pltpu.SemaphoreType.DMA,
          pltpu.SemaphoreType.DMA,
      ],
  )
  def sc_kernel(x_hbm, lab_hbm, cen_hbm, outd, outc, outx,
                idx_v, x_bufs, c_bufs, ad_v, ac_v, ax_v, *sems):
    semx = sems[:NBUF]
    semc = sems[NBUF:]
    wid = lax.axis_index("s") * NC + lax.axis_index("c")
    base = pl.multiple_of(wid * BPW, BPW)
    pltpu.sync_copy(lab_hbm.at[pl.ds(base, BPW)], idx_v)

    def start(k, b):
      off = pl.multiple_of(base + k * ROW_CHUNK, ROW_CHUNK)
      pltpu.async_copy(
          x_hbm.at[pl.ds(off, ROW_CHUNK)], x_bufs.at[b], semx[b])
      ioff = pl.multiple_of(k * ROW_CHUNK, ROW_CHUNK)
      pltpu.async_copy(
          cen_hbm.at[idx_v.at[pl.ds(ioff, ROW_CHUNK)]],
          c_bufs.at[b], semc[b])

    def wait(b):
      pltpu.make_async_copy(
          x_hbm.at[pl.ds(0, ROW_CHUNK)], x_bufs.at[b], semx[b]).wait()
      pltpu.make_async_copy(
          cen_hbm.at[pl.ds(0, ROW_CHUNK)], c_bufs.at[b], semc[b]).wait()

    for b in range(NBUF):
      start(b, b)

    def outer(g, _):
      for b in range(NBUF):
        k = g * NBUF + b
        wait(b)
        x_v = x_bufs.at[b]
        c_v = c_bufs.at[b]

        @plsc.parallel_loop(0, ROW_CHUNK, unroll=8)
        def row_body(r):
          zero = jnp.zeros((LANES,), jnp.float32)
          pd = [zero, zero]
          pc = [zero, zero]
          px = [zero, zero]
          for v in range(NVEC):
            xv = x_v[r, pl.ds(v * LANES, LANES)]
            cv = c_v[r, pl.ds(v * LANES, LANES)]
            j = v % 2
            pd[j] = pd[j] + xv * cv
            pc[j] = pc[j] + cv * cv
            px[j] = px[j] + xv * xv
          q = k * (ROW_CHUNK // 8) + r // 8
          s = (r % 8) * LANES
          ad_v[q, pl.ds(s, LANES)] = pd[0] + pd[1]
          ac_v[q, pl.ds(s, LANES)] = pc[0] + pc[1]
          ax_v[q, pl.ds(s, LANES)] = px[0] + px[1]

        start(jnp.minimum(k + NBUF, NCHUNK - 1), b)
      return 0

    lax.fori_loop(0, NCHUNK // NBUF, outer, 0, unroll=False)
    for b in range(NBUF):
      wait(b)

    orow = pl.multiple_of(base // 8, 8)
    pltpu.sync_copy(ad_v, outd.at[pl.ds(orow, BPW // 8)])
    pltpu.sync_copy(ac_v, outc.at[pl.ds(orow, BPW // 8)])
    pltpu.sync_copy(ax_v, outx.at[pl.ds(orow, BPW // 8)])

  return sc_kernel(x, labels_i32, centers)


def _tc_finish(dp, cp, xp, sel):
  def body(d_ref, c_ref, x_ref, sel_ref, o_ref):
    sel_m = sel_ref[...]
    dot = lax.dot_general(d_ref[...], sel_m, (((1,), (0,)), ((), ())),
                          preferred_element_type=jnp.float32)
    c2 = lax.dot_general(c_ref[...], sel_m, (((1,), (0,)), ((), ())),
                         preferred_element_type=jnp.float32)
    x2 = lax.dot_general(x_ref[...], sel_m, (((1,), (0,)), ((), ())),
                         preferred_element_type=jnp.float32)
    cos = dot / jnp.maximum(jnp.sqrt(x2) * jnp.sqrt(c2), 1e-8)
    dist = jnp.exp(-3.5 * (cos - 1.0))
    o_ref[0, 0] = jnp.sum(dist) * (1.0 / BATCH)

  out = pl.pallas_call(
      body,
      in_specs=[pl.BlockSpec((OUT_ROWS, 128), lambda: (0, 0))] * 3
      + [pl.BlockSpec((128, 8), lambda: (0, 0))],
      out_specs=pl.BlockSpec(memory_space=pltpu.SMEM),
      out_shape=jax.ShapeDtypeStruct((1, 1), jnp.float32),
  )(dp, cp, xp, sel)
  return out[0, 0]


@jax.jit
def kernel(x, labels, centers):
  labels_i32 = labels.astype(jnp.int32)
  dp, cp, xp = _sc_partials(x, labels_i32, centers)
  sel = (jnp.arange(128, dtype=jnp.int32)[:, None] // LANES
         == jnp.arange(8, dtype=jnp.int32)[None, :]).astype(jnp.float32)
  return _tc_finish(dp, cp, xp, sel)
